# baseline (speedup 1.0000x reference)
import jax
import jax.numpy as jnp
from jax import lax
from jax.experimental import pallas as pl
from jax.experimental.pallas import tpu as pltpu
from jax.experimental.pallas import tpu_sc as plsc

_LANES = 16
_NC, _NS = 2, 16
_NW = _NC * _NS
_ROWS = 64 * 128
_B = 8
_GROUPS = _ROWS // _B
_GPW = _GROUPS // _NW
_PRE = 8


def _fold_spec(oh, ow, kh, kw, sh, sw, ph, pw, dh, dw, ntap, L):
    assert sw == 1, "all three folds have unit output-width stride"
    Lh = (oh + 2 * ph - dh * (kh - 1) - 1) // sh + 1
    Lw = (ow + 2 * pw - dw * (kw - 1) - 1) // sw + 1
    assert ntap == kh * kw and L == Lh * Lw
    slab = ntap * L
    olen = oh * ow
    rows = [[] for _ in range(oh)]
    for ki in range(kh):
        for kj in range(kw):
            for lh in range(Lh):
                r = lh * sh + ki * dh - ph
                if r < 0 or r >= oh:
                    continue
                c0 = kj * dw - pw
                s = max(0, c0)
                e = min(Lw + c0, ow)
                if e <= s:
                    continue
                rows[r].append((((ki * kw + kj) * Lh + lh) * Lw - c0, s, e))
    vecs = []
    for r in range(oh):
        for k in range(0, ow, _LANES):
            contribs = []
            for src0, s, e in rows[r]:
                a = max(s - k, 0)
                b = min(e - k, _LANES)
                if b > a:
                    contribs.append((src0 + k, a, b))
            assert contribs
            vecs.append((r * ow + k, contribs))
    tail = olen - vecs[-1][0]
    return dict(slab=slab, olen=olen, tail=tail, vecs=vecs, ntap=ntap, L=L)


_SPECS = (
    _fold_spec(22, 22, 3, 3, 1, 1, 0, 0, 1, 1, 9, 400),
    _fold_spec(17, 18, 2, 4, 2, 1, 2, 2, 1, 1, 8, 190),
    _fold_spec(5, 11, 2, 3, 1, 1, 2, 4, 1, 2, 6, 120),
)
_OGUARD = (12, 16, 8)


def _sc_fold_kernel(xh, yh, zh, oxh, oyh, ozh,
                    bx0, by0, bx1, by1, bz, slx, sly,
                    obx, oby, obz,
                    si0, si1, siz, so):
    wid = lax.axis_index("s") * _NC + lax.axis_index("c")
    g0 = wid * _GPW
    in_slots = ((bx0, by0), (bx1, by1))
    in_sems = (si0, si1)
    obufs = (obx, oby, obz)

    iota = lax.iota(jnp.int32, _LANES)
    mask_keys = sorted({(a, b)
                        for spec in _SPECS
                        for _, contribs in spec["vecs"]
                        for (_, a, b) in contribs if (a, b) != (0, _LANES)})
    masks = {ab: (iota >= ab[0]) & (iota < ab[1]) for ab in mask_keys}

    def in_copy(g, slot):
        for hbm, buf, spec in zip((xh, yh), in_slots[slot], _SPECS[:2]):
            nr = _B * spec["ntap"]
            yield pltpu.make_async_copy(
                hbm.at[pl.ds(g * nr, nr), :], buf, in_sems[slot])

    def z_in_copy(g):
        sz = _B * _SPECS[2]["slab"]
        return pltpu.make_async_copy(
            zh.at[pl.ds(g * sz, sz)], bz.at[pl.ds(0, sz)], siz)

    def out_copy(g):
        for hbm, buf, spec in zip((oxh, oyh, ozh), obufs, _SPECS):
            sz = _B * spec["olen"]
            yield pltpu.make_async_copy(
                buf.at[pl.ds(0, sz)], hbm.at[pl.ds(g * sz, sz)], so)

    def compute_sample(slot, i):
        for buf2d, slab1d, spec in zip(in_slots[slot], (slx, sly), _SPECS[:2]):
            ntap, L = spec["ntap"], spec["L"]
            for tap in range(ntap):
                row = i * ntap + tap
                dst = _PRE + tap * L
                for c in range(0, L - _LANES + 1, _LANES):
                    slab1d[pl.ds(dst + c, _LANES)] = buf2d[row, pl.ds(c, _LANES)]
                if L % _LANES:
                    slab1d[pl.ds(dst + L - _LANES, _LANES)] = (
                        buf2d[row, pl.ds(L - _LANES, _LANES)])
        srcs = (slx, sly, bz)
        bases = (_PRE, _PRE, i * _SPECS[2]["slab"])
        for buf_i, base, buf_o, spec in zip(srcs, bases, obufs, _SPECS):
            obase = i * spec["olen"]
            pend = []

            def flush(pend):
                for o2, a2 in pend:
                    buf_o[pl.ds(obase + o2, _LANES)] = a2

            for off, contribs in spec["vecs"]:
                acc = None
                for lo, a, b in contribs:
                    v = buf_i[pl.ds(base + lo, _LANES)]
                    if (a, b) != (0, _LANES):
                        v = jnp.where(masks[(a, b)], v, 0.0)
                    acc = v if acc is None else acc + v
                pend.append((off, acc))
                if len(pend) == 4:
                    flush(pend)
                    pend = []
            flush(pend)

    for c in in_copy(g0, 0):
        c.start()
    for c in in_copy(g0 + 1, 1):
        c.start()
    z_in_copy(g0).start()

    @pl.loop(0, _GPW, step=2)
    def _(t):
        for slot in (0, 1):
            g = g0 + t + slot
            for c in in_copy(g, slot):
                c.wait()
            z_in_copy(g).wait()

            @pl.when(g > g0)
            def _():
                for c in out_copy(g - 1):
                    c.wait()

            @pl.loop(0, _B)
            def _(i):
                compute_sample(slot, i)

            for c in out_copy(g):
                c.start()

            @pl.when(g + 1 < g0 + _GPW)
            def _():
                z_in_copy(g + 1).start()

            @pl.when(t + slot + 2 < _GPW)
            def _():
                for c in in_copy(g + 2, slot):
                    c.start()

    for c in out_copy(g0 + _GPW - 1):
        c.wait()


@jax.jit
def kernel(x, y, z):
    run = pl.kernel(
        _sc_fold_kernel,
        out_type=tuple(
            jax.ShapeDtypeStruct((_ROWS * s["olen"],), jnp.float32)
            for s in _SPECS),
        mesh=plsc.VectorSubcoreMesh(core_axis_name="c", subcore_axis_name="s"),
        scratch_types=(
            pltpu.VMEM((_B * 9, 400), jnp.float32),
            pltpu.VMEM((_B * 8, 190), jnp.float32),
            pltpu.VMEM((_B * 9, 400), jnp.float32),
            pltpu.VMEM((_B * 8, 190), jnp.float32),
            pltpu.VMEM((_B * _SPECS[2]["slab"] + _LANES,), jnp.float32),
            pltpu.VMEM((_PRE + _SPECS[0]["slab"] + _LANES,), jnp.float32),
            pltpu.VMEM((_PRE + _SPECS[1]["slab"] + _LANES,), jnp.float32),
            pltpu.VMEM((_B * _SPECS[0]["olen"] + _OGUARD[0],), jnp.float32),
            pltpu.VMEM((_B * _SPECS[1]["olen"] + _OGUARD[1],), jnp.float32),
            pltpu.VMEM((_B * _SPECS[2]["olen"] + _OGUARD[2],), jnp.float32),
            pltpu.SemaphoreType.DMA,
            pltpu.SemaphoreType.DMA,
            pltpu.SemaphoreType.DMA,
            pltpu.SemaphoreType.DMA,
        ),
    )
    xo, yo, zo = run(x.reshape(64 * 1152, 400),
                     y.reshape(64 * 1024, 190),
                     z.reshape(-1))
    return (xo.reshape(64, 128, 22, 22),
            yo.reshape(64, 128, 17, 18),
            zo.reshape(64, 128, 5, 11))
